# baseline (speedup 1.0000x reference)
import functools

import jax
import jax.numpy as jnp
from jax import lax
from jax.experimental import pallas as pl
from jax.experimental.pallas import tpu as pltpu
from jax.experimental.pallas import tpu_sc as plsc

_N = 10000
_NP = 10240
_NP2 = 12288
_E = 160000
_EP = 163840
_CW = 128
_NC = 2
_NS = 16
_NW = _NC * _NS
_ESTEPS = _EP // _NW // _CW
_S0 = 40
_S1 = 40
_EROWS = _EP // _CW
_EPAD = (_EROWS + _S0 - _S1) * _CW
_GSTEPS = _NP2 // _NW // _CW
_SROWS = _NP // _NS
_BN = 512
_GRID = _NP // _BN
_HID = 512
_OUT = 256


def _sc_mesh():
    return plsc.VectorSubcoreMesh(core_axis_name="c", subcore_axis_name="s",
                                  num_cores=_NC, num_subcores=_NS)


@functools.cache
def _get_emb_kernel():
    def body(deg_idx_hbm, id_idx_hbm, deg_emb_hbm, id_emb_hbm,
             df_hbm, if_hbm, degv, nidv, rows):
        cid = lax.axis_index("c")
        sid = lax.axis_index("s")
        wid = cid * _NS + sid
        pltpu.sync_copy(deg_idx_hbm, degv)
        pltpu.sync_copy(id_idx_hbm, nidv)
        for s in range(_GSTEPS):
            base = wid * _GSTEPS * _CW + s * _CW
            row = wid * _GSTEPS + s
            pltpu.sync_copy(deg_emb_hbm.at[degv.at[row]], rows)
            pltpu.sync_copy(rows, df_hbm.at[pl.ds(base, _CW)])
            pltpu.sync_copy(id_emb_hbm.at[nidv.at[row]], rows)
            pltpu.sync_copy(rows, if_hbm.at[pl.ds(base, _CW)])

    return pl.kernel(
        body,
        out_type=(
            jax.ShapeDtypeStruct((_NP2, _CW), jnp.float32),
            jax.ShapeDtypeStruct((_NP2, _CW), jnp.float32),
        ),
        mesh=_sc_mesh(),
        scratch_types=(
            pltpu.VMEM((_NW * _GSTEPS, _CW), jnp.int32),
            pltpu.VMEM((_NW * _GSTEPS, _CW), jnp.int32),
            pltpu.VMEM((_CW, _CW), jnp.float32),
        ),
        name="sc_emb_gather",
    )


@functools.cache
def _make_agg(num_chunks, with_ones):
    n_out = num_chunks + (1 if with_ones else 0)

    def body(*refs):
        xs = refs[:num_chunks]
        src_hbm, dst_hbm, zeros_hbm, ones_hbm = refs[num_chunks:num_chunks + 4]
        outs = refs[num_chunks + 4:num_chunks + 4 + n_out]
        srcv, dstv, rows2, slab, gsem, ssem = refs[num_chunks + 4 + n_out:]

        cid = lax.axis_index("c")
        sid = lax.axis_index("s")
        base = lax.select(cid == 0, sid * _S0, _NS * _S0 + sid * _S1)
        nsteps = lax.select(cid == 0, jnp.int32(_S0), jnp.int32(_S1))
        pltpu.sync_copy(src_hbm.at[pl.ds(base, _S0)], srcv)
        pltpu.sync_copy(dst_hbm.at[pl.ds(base, _S0)], dstv)
        row0 = sid * _SROWS

        def zero_slab():
            for k in range(_SROWS // _CW):
                pltpu.sync_copy(rows2.at[0], slab.at[pl.ds(row0 + k * _CW,
                                                           _CW)])

        def copy_out(o):
            pltpu.sync_copy(slab.at[pl.ds(row0, _SROWS)],
                            o.at[cid, pl.ds(row0, _SROWS)])

        for c in range(num_chunks):
            pltpu.sync_copy(zeros_hbm, rows2.at[0])
            zero_slab()
            plsc.subcore_barrier()

            def step(j, carry, _c=c):
                slot = lax.rem(j, 2)
                nslot = lax.rem(j + 1, 2)

                @pl.when(j + 1 < nsteps)
                def _():
                    @pl.when(j >= 1)
                    def _():
                        pltpu.make_async_copy(rows2.at[nslot],
                                              slab.at[dstv.at[j - 1]],
                                              ssem).wait()

                    pltpu.make_async_copy(xs[_c].at[srcv.at[j + 1]],
                                          rows2.at[nslot], gsem).start()

                pltpu.make_async_copy(xs[_c].at[srcv.at[j]],
                                      rows2.at[slot], gsem).wait()
                pltpu.async_copy(rows2.at[slot], slab.at[dstv.at[j]], ssem,
                                 add=True)
                return carry

            pltpu.make_async_copy(xs[c].at[srcv.at[0]], rows2.at[0],
                                  gsem).start()
            lax.fori_loop(0, nsteps, step, 0)
            pltpu.make_async_copy(rows2.at[0],
                                  slab.at[dstv.at[nsteps - 2]],
                                  ssem).wait()
            pltpu.make_async_copy(rows2.at[1],
                                  slab.at[dstv.at[nsteps - 1]],
                                  ssem).wait()
            plsc.subcore_barrier()
            copy_out(outs[c])
            plsc.subcore_barrier()

        if with_ones:
            pltpu.sync_copy(zeros_hbm, rows2.at[0])
            zero_slab()
            pltpu.sync_copy(ones_hbm, rows2.at[0])
            plsc.subcore_barrier()

            def step1(j, carry):
                pltpu.sync_copy(rows2.at[0], slab.at[dstv.at[j]], add=True)
                return carry

            lax.fori_loop(0, nsteps, step1, 0)
            plsc.subcore_barrier()
            copy_out(outs[-1])

    return pl.kernel(
        body,
        out_type=tuple(
            jax.ShapeDtypeStruct((_NC, _NP, _CW), jnp.float32)
            for _ in range(n_out)),
        mesh=_sc_mesh(),
        scratch_types=(
            pltpu.VMEM((_S0, _CW), jnp.int32),
            pltpu.VMEM((_S0, _CW), jnp.int32),
            pltpu.VMEM((2, _CW, _CW), jnp.float32),
            pltpu.VMEM_SHARED((_NP, _CW), jnp.float32),
            pltpu.SemaphoreType.DMA,
            pltpu.SemaphoreType.DMA,
        ),
        name=f"sc_agg_{num_chunks}{'_deg' if with_ones else ''}",
    )


def _inv_deg(pdeg_r):
    deg = pdeg_r[0, :, :1] + pdeg_r[1, :, :1]
    return 1.0 / jnp.maximum(deg, 1.0)


def _l0_body(df_r, if_r, pd_r, pi_r, pdeg_r,
             wsd_r, wsi_r, wnd_r, wni_r, b_r,
             o0, o1, o2, o3):
    inv = _inv_deg(pdeg_r)
    h = jnp.dot(df_r[...], wsd_r[...], preferred_element_type=jnp.float32)
    h = h + jnp.dot(if_r[...], wsi_r[...], preferred_element_type=jnp.float32)
    h = h + jnp.dot((pd_r[0] + pd_r[1]) * inv, wnd_r[...],
                    preferred_element_type=jnp.float32)
    h = h + jnp.dot((pi_r[0] + pi_r[1]) * inv, wni_r[...],
                    preferred_element_type=jnp.float32)
    h = h + b_r[...]
    o0[...] = h[:, 0 * _CW:1 * _CW]
    o1[...] = h[:, 1 * _CW:2 * _CW]
    o2[...] = h[:, 2 * _CW:3 * _CW]
    o3[...] = h[:, 3 * _CW:4 * _CW]


def _l1_body(h0, h1, h2, h3, p0, p1, p2, p3, pdeg_r,
             ws0, ws1, ws2, ws3, wn0, wn1, wn2, wn3, b1_r,
             ws2t_r, wn2t_r, b2_r,
             hs_o, hn0_o, hn1_o):
    inv = _inv_deg(pdeg_r)
    hs_in = (h0, h1, h2, h3)
    ps = (p0, p1, p2, p3)
    ws = (ws0, ws1, ws2, ws3)
    wn = (wn0, wn1, wn2, wn3)
    h = b1_r[...] + jnp.zeros((_BN, _HID), jnp.float32)
    for c in range(4):
        h = h + jnp.dot(hs_in[c][...], ws[c][...],
                        preferred_element_type=jnp.float32)
        h = h + jnp.dot((ps[c][0] + ps[c][1]) * inv, wn[c][...],
                        preferred_element_type=jnp.float32)
    hs = jnp.dot(h, ws2t_r[...], preferred_element_type=jnp.float32) + b2_r[...]
    hn = jnp.dot(h, wn2t_r[...], preferred_element_type=jnp.float32)
    hs_o[...] = hs
    hn0_o[...] = hn[:, :_CW]
    hn1_o[...] = hn[:, _CW:]


def _fin_body(hs_r, q0, q1, pdeg_r, out_o):
    inv = _inv_deg(pdeg_r)
    a0 = (q0[0] + q0[1]) * inv
    a1 = (q1[0] + q1[1]) * inv
    out_o[...] = hs_r[...] + jnp.concatenate([a0, a1], axis=1)


def _row_spec(w):
    return pl.BlockSpec((_BN, w), lambda i: (i, 0))


def _part_spec():
    return pl.BlockSpec((_NC, _BN, _CW), lambda i: (0, i, 0))


def _full_spec(shape):
    nd = len(shape)
    return pl.BlockSpec(shape, lambda i: (0,) * nd)


def _tc_layer0(df, if_, p_df, p_if, p_deg, wsd, wsi, wnd, wni, b0):
    return pl.pallas_call(
        _l0_body,
        grid=(_GRID,),
        in_specs=[
            _row_spec(_CW), _row_spec(_CW),
            _part_spec(), _part_spec(), _part_spec(),
            _full_spec((_CW, _HID)), _full_spec((_CW, _HID)),
            _full_spec((_CW, _HID)), _full_spec((_CW, _HID)),
            _full_spec((1, _HID)),
        ],
        out_specs=[_row_spec(_CW)] * 4,
        out_shape=[jax.ShapeDtypeStruct((_NP, _CW), jnp.float32)] * 4,
    )(df, if_, p_df, p_if, p_deg, wsd, wsi, wnd, wni, b0)


def _tc_layer12(h1c, p1c, p_deg, ws1c, wn1c, b1, ws2t, wn2t, b2):
    return pl.pallas_call(
        _l1_body,
        grid=(_GRID,),
        in_specs=(
            [_row_spec(_CW)] * 4
            + [_part_spec()] * 5
            + [_full_spec((_CW, _HID))] * 8
            + [_full_spec((1, _HID)),
               _full_spec((_HID, _OUT)), _full_spec((_HID, _OUT)),
               _full_spec((1, _OUT))]
        ),
        out_specs=[_row_spec(_OUT), _row_spec(_CW), _row_spec(_CW)],
        out_shape=[
            jax.ShapeDtypeStruct((_NP, _OUT), jnp.float32),
            jax.ShapeDtypeStruct((_NP, _CW), jnp.float32),
            jax.ShapeDtypeStruct((_NP, _CW), jnp.float32),
        ],
    )(*h1c, *p1c, p_deg, *ws1c, *wn1c, b1, ws2t, wn2t, b2)


def _tc_final(hs, q0, q1, p_deg):
    return pl.pallas_call(
        _fin_body,
        grid=(_GRID,),
        in_specs=[_row_spec(_OUT), _part_spec(), _part_spec(), _part_spec()],
        out_specs=_row_spec(_OUT),
        out_shape=jax.ShapeDtypeStruct((_NP, _OUT), jnp.float32),
    )(hs, q0, q1, p_deg)


@jax.jit
def _run(degree, node_ids, edge_index, deg_emb, id_emb,
         W_self_0, W_neigh_0, b_0,
         W_self_1, W_neigh_1, b_1,
         W_self_2, W_neigh_2, b_2):
    i32 = jnp.int32
    degree = degree.astype(i32)
    node_ids = node_ids.astype(i32)
    src = edge_index[0].astype(i32)
    dst = edge_index[1].astype(i32)
    order = jnp.argsort(src)
    src = jnp.take(src, order)
    dst = jnp.take(dst, order)

    deg_idx = jnp.concatenate(
        [degree, jnp.zeros((_NP2 - _N,), i32)]).reshape(_NW * _GSTEPS, _CW)
    id_idx = jnp.concatenate(
        [node_ids, jnp.zeros((_NP2 - _N,), i32)]).reshape(_NW * _GSTEPS, _CW)
    src_p = jnp.concatenate(
        [src, jnp.zeros((_EPAD - _E,), i32)]).reshape(-1, _CW)
    dst_p = jnp.concatenate(
        [dst, jnp.full((_EPAD - _E,), _N, i32)]).reshape(-1, _CW)
    zeros_c = jnp.zeros((_CW, _CW), jnp.float32)
    ones_c = jnp.ones((_CW, _CW), jnp.float32)

    df, if_ = _get_emb_kernel()(deg_idx, id_idx, deg_emb, id_emb)

    p_df, p_if, p_deg = _make_agg(2, True)(df, if_, src_p, dst_p,
                                           zeros_c, ones_c)

    wsd = W_self_0[:, :_CW].T
    wsi = W_self_0[:, _CW:].T
    wnd = W_neigh_0[:, :_CW].T
    wni = W_neigh_0[:, _CW:].T
    h1c = _tc_layer0(df, if_, p_df, p_if, p_deg,
                     wsd, wsi, wnd, wni, b_0.reshape(1, _HID))

    p1c = _make_agg(4, False)(*h1c, src_p, dst_p, zeros_c, ones_c)

    ws1c = tuple(W_self_1[:, c * _CW:(c + 1) * _CW].T for c in range(4))
    wn1c = tuple(W_neigh_1[:, c * _CW:(c + 1) * _CW].T for c in range(4))
    hs, hn0, hn1 = _tc_layer12(h1c, p1c, p_deg, ws1c, wn1c,
                               b_1.reshape(1, _HID),
                               W_self_2.T, W_neigh_2.T,
                               b_2.reshape(1, _OUT))

    q0, q1 = _make_agg(2, False)(hn0, hn1, src_p, dst_p, zeros_c, ones_c)

    out = _tc_final(hs, q0, q1, p_deg)
    return out[:_N]


def kernel(degree, node_ids, edge_index, deg_emb, id_emb,
           W_self_0, W_neigh_0, b_0,
           W_self_1, W_neigh_1, b_1,
           W_self_2, W_neigh_2, b_2):
    return _run(degree, node_ids, edge_index, deg_emb, id_emb,
                W_self_0, W_neigh_0, b_0,
                W_self_1, W_neigh_1, b_1,
                W_self_2, W_neigh_2, b_2)


# speedup vs baseline: 1.1254x; 1.1254x over previous
import functools

import jax
import jax.numpy as jnp
from jax import lax
from jax.experimental import pallas as pl
from jax.experimental.pallas import tpu as pltpu
from jax.experimental.pallas import tpu_sc as plsc

_N = 10000
_NP = 10240
_NP2 = 12288
_E = 160000
_EP = 163840
_CW = 128
_NC = 2
_NS = 16
_NW = _NC * _NS
_ESTEPS = _EP // _NW // _CW
_S0 = 40
_S1 = 40
_EROWS = _EP // _CW
_EPAD = (_EROWS + _S0 - _S1) * _CW
_GSTEPS = _NP2 // _NW // _CW
_SROWS = _NP // _NS
_BN = 512
_GRID = _NP // _BN
_HID = 512
_OUT = 256


def _sc_mesh():
    return plsc.VectorSubcoreMesh(core_axis_name="c", subcore_axis_name="s",
                                  num_cores=_NC, num_subcores=_NS)


@functools.cache
def _get_emb_kernel():
    def body(deg_idx_hbm, id_idx_hbm, deg_emb_hbm, id_emb_hbm,
             df_hbm, if_hbm, degv, nidv, rows):
        cid = lax.axis_index("c")
        sid = lax.axis_index("s")
        wid = cid * _NS + sid
        pltpu.sync_copy(deg_idx_hbm, degv)
        pltpu.sync_copy(id_idx_hbm, nidv)
        for s in range(_GSTEPS):
            base = wid * _GSTEPS * _CW + s * _CW
            row = wid * _GSTEPS + s
            pltpu.sync_copy(deg_emb_hbm.at[degv.at[row]], rows)
            pltpu.sync_copy(rows, df_hbm.at[pl.ds(base, _CW)])
            pltpu.sync_copy(id_emb_hbm.at[nidv.at[row]], rows)
            pltpu.sync_copy(rows, if_hbm.at[pl.ds(base, _CW)])

    return pl.kernel(
        body,
        out_type=(
            jax.ShapeDtypeStruct((_NP2, _CW), jnp.float32),
            jax.ShapeDtypeStruct((_NP2, _CW), jnp.float32),
        ),
        mesh=_sc_mesh(),
        scratch_types=(
            pltpu.VMEM((_NW * _GSTEPS, _CW), jnp.int32),
            pltpu.VMEM((_NW * _GSTEPS, _CW), jnp.int32),
            pltpu.VMEM((_CW, _CW), jnp.float32),
        ),
        name="sc_emb_gather",
    )


@functools.cache
def _make_agg(num_chunks, with_ones):
    n_out = num_chunks + (1 if with_ones else 0)

    def body(*refs):
        xs = refs[:num_chunks]
        src_hbm, dst_hbm, zeros_hbm, ones_hbm = refs[num_chunks:num_chunks + 4]
        outs = refs[num_chunks + 4:num_chunks + 4 + n_out]
        srcv, dstv, rows2, slab, gsem, ssem = refs[num_chunks + 4 + n_out:]

        cid = lax.axis_index("c")
        sid = lax.axis_index("s")
        base = lax.select(cid == 0, sid * _S0, _NS * _S0 + sid * _S1)
        nsteps = lax.select(cid == 0, jnp.int32(_S0), jnp.int32(_S1))
        pltpu.sync_copy(src_hbm.at[pl.ds(base, _S0)], srcv)
        pltpu.sync_copy(dst_hbm.at[pl.ds(base, _S0)], dstv)
        row0 = sid * _SROWS

        def zero_slab():
            for k in range(_SROWS // _CW):
                pltpu.sync_copy(rows2.at[0], slab.at[pl.ds(row0 + k * _CW,
                                                           _CW)])

        def copy_out(o):
            pltpu.sync_copy(slab.at[pl.ds(row0, _SROWS)],
                            o.at[cid, pl.ds(row0, _SROWS)])

        def start_gather(xref, j, slot):
            pltpu.make_async_copy(xref.at[srcv.at[j, pl.ds(0, 64)]],
                                  rows2.at[slot, pl.ds(0, 64)], gsem).start()
            pltpu.make_async_copy(xref.at[srcv.at[j, pl.ds(64, 64)]],
                                  rows2.at[slot, pl.ds(64, 64)], gsem).start()

        def wait_gather(xref, j, slot):
            pltpu.make_async_copy(xref.at[srcv.at[j, pl.ds(0, 64)]],
                                  rows2.at[slot, pl.ds(0, 64)], gsem).wait()
            pltpu.make_async_copy(xref.at[srcv.at[j, pl.ds(64, 64)]],
                                  rows2.at[slot, pl.ds(64, 64)], gsem).wait()

        for c in range(num_chunks):
            pltpu.sync_copy(zeros_hbm, rows2.at[0])
            zero_slab()
            plsc.subcore_barrier()

            def step(j, carry, _c=c):
                slot = lax.rem(j, 2)
                nslot = lax.rem(j + 1, 2)

                @pl.when(j + 1 < nsteps)
                def _():
                    @pl.when(j >= 1)
                    def _():
                        pltpu.make_async_copy(rows2.at[nslot],
                                              slab.at[dstv.at[j - 1]],
                                              ssem).wait()

                    start_gather(xs[_c], j + 1, nslot)

                wait_gather(xs[_c], j, slot)
                pltpu.async_copy(rows2.at[slot], slab.at[dstv.at[j]], ssem,
                                 add=True)
                return carry

            start_gather(xs[c], 0, 0)
            lax.fori_loop(0, nsteps, step, 0)
            pltpu.make_async_copy(rows2.at[0],
                                  slab.at[dstv.at[nsteps - 2]],
                                  ssem).wait()
            pltpu.make_async_copy(rows2.at[1],
                                  slab.at[dstv.at[nsteps - 1]],
                                  ssem).wait()
            plsc.subcore_barrier()
            copy_out(outs[c])
            plsc.subcore_barrier()

        if with_ones:
            pltpu.sync_copy(zeros_hbm, rows2.at[0])
            zero_slab()
            pltpu.sync_copy(ones_hbm, rows2.at[0])
            plsc.subcore_barrier()

            def step1(j, carry):
                pltpu.sync_copy(rows2.at[0], slab.at[dstv.at[j]], add=True)
                return carry

            lax.fori_loop(0, nsteps, step1, 0)
            plsc.subcore_barrier()
            copy_out(outs[-1])

    return pl.kernel(
        body,
        out_type=tuple(
            jax.ShapeDtypeStruct((_NC, _NP, _CW), jnp.float32)
            for _ in range(n_out)),
        mesh=_sc_mesh(),
        scratch_types=(
            pltpu.VMEM((_S0, _CW), jnp.int32),
            pltpu.VMEM((_S0, _CW), jnp.int32),
            pltpu.VMEM((2, _CW, _CW), jnp.float32),
            pltpu.VMEM_SHARED((_NP, _CW), jnp.float32),
            pltpu.SemaphoreType.DMA,
            pltpu.SemaphoreType.DMA,
        ),
        name=f"sc_agg_{num_chunks}{'_deg' if with_ones else ''}",
    )


def _inv_deg(pdeg_r):
    deg = pdeg_r[0, :, :1] + pdeg_r[1, :, :1]
    return 1.0 / jnp.maximum(deg, 1.0)


def _l0_body(df_r, if_r, pd_r, pi_r, pdeg_r,
             wsd_r, wsi_r, wnd_r, wni_r, b_r,
             o0, o1, o2, o3):
    inv = _inv_deg(pdeg_r)
    h = jnp.dot(df_r[...], wsd_r[...], preferred_element_type=jnp.float32)
    h = h + jnp.dot(if_r[...], wsi_r[...], preferred_element_type=jnp.float32)
    h = h + jnp.dot((pd_r[0] + pd_r[1]) * inv, wnd_r[...],
                    preferred_element_type=jnp.float32)
    h = h + jnp.dot((pi_r[0] + pi_r[1]) * inv, wni_r[...],
                    preferred_element_type=jnp.float32)
    h = h + b_r[...]
    o0[...] = h[:, 0 * _CW:1 * _CW]
    o1[...] = h[:, 1 * _CW:2 * _CW]
    o2[...] = h[:, 2 * _CW:3 * _CW]
    o3[...] = h[:, 3 * _CW:4 * _CW]


def _l1_body(h0, h1, h2, h3, p0, p1, p2, p3, pdeg_r,
             ws0, ws1, ws2, ws3, wn0, wn1, wn2, wn3, b1_r,
             ws2t_r, wn2t_r, b2_r,
             hs_o, hn0_o, hn1_o):
    inv = _inv_deg(pdeg_r)
    hs_in = (h0, h1, h2, h3)
    ps = (p0, p1, p2, p3)
    ws = (ws0, ws1, ws2, ws3)
    wn = (wn0, wn1, wn2, wn3)
    h = b1_r[...] + jnp.zeros((_BN, _HID), jnp.float32)
    for c in range(4):
        h = h + jnp.dot(hs_in[c][...], ws[c][...],
                        preferred_element_type=jnp.float32)
        h = h + jnp.dot((ps[c][0] + ps[c][1]) * inv, wn[c][...],
                        preferred_element_type=jnp.float32)
    hs = jnp.dot(h, ws2t_r[...], preferred_element_type=jnp.float32) + b2_r[...]
    hn = jnp.dot(h, wn2t_r[...], preferred_element_type=jnp.float32)
    hs_o[...] = hs
    hn0_o[...] = hn[:, :_CW]
    hn1_o[...] = hn[:, _CW:]


def _fin_body(hs_r, q0, q1, pdeg_r, out_o):
    inv = _inv_deg(pdeg_r)
    a0 = (q0[0] + q0[1]) * inv
    a1 = (q1[0] + q1[1]) * inv
    out_o[...] = hs_r[...] + jnp.concatenate([a0, a1], axis=1)


def _row_spec(w):
    return pl.BlockSpec((_BN, w), lambda i: (i, 0))


def _part_spec():
    return pl.BlockSpec((_NC, _BN, _CW), lambda i: (0, i, 0))


def _full_spec(shape):
    nd = len(shape)
    return pl.BlockSpec(shape, lambda i: (0,) * nd)


def _tc_layer0(df, if_, p_df, p_if, p_deg, wsd, wsi, wnd, wni, b0):
    return pl.pallas_call(
        _l0_body,
        grid=(_GRID,),
        in_specs=[
            _row_spec(_CW), _row_spec(_CW),
            _part_spec(), _part_spec(), _part_spec(),
            _full_spec((_CW, _HID)), _full_spec((_CW, _HID)),
            _full_spec((_CW, _HID)), _full_spec((_CW, _HID)),
            _full_spec((1, _HID)),
        ],
        out_specs=[_row_spec(_CW)] * 4,
        out_shape=[jax.ShapeDtypeStruct((_NP, _CW), jnp.float32)] * 4,
    )(df, if_, p_df, p_if, p_deg, wsd, wsi, wnd, wni, b0)


def _tc_layer12(h1c, p1c, p_deg, ws1c, wn1c, b1, ws2t, wn2t, b2):
    return pl.pallas_call(
        _l1_body,
        grid=(_GRID,),
        in_specs=(
            [_row_spec(_CW)] * 4
            + [_part_spec()] * 5
            + [_full_spec((_CW, _HID))] * 8
            + [_full_spec((1, _HID)),
               _full_spec((_HID, _OUT)), _full_spec((_HID, _OUT)),
               _full_spec((1, _OUT))]
        ),
        out_specs=[_row_spec(_OUT), _row_spec(_CW), _row_spec(_CW)],
        out_shape=[
            jax.ShapeDtypeStruct((_NP, _OUT), jnp.float32),
            jax.ShapeDtypeStruct((_NP, _CW), jnp.float32),
            jax.ShapeDtypeStruct((_NP, _CW), jnp.float32),
        ],
    )(*h1c, *p1c, p_deg, *ws1c, *wn1c, b1, ws2t, wn2t, b2)


def _tc_final(hs, q0, q1, p_deg):
    return pl.pallas_call(
        _fin_body,
        grid=(_GRID,),
        in_specs=[_row_spec(_OUT), _part_spec(), _part_spec(), _part_spec()],
        out_specs=_row_spec(_OUT),
        out_shape=jax.ShapeDtypeStruct((_NP, _OUT), jnp.float32),
    )(hs, q0, q1, p_deg)


@jax.jit
def _run(degree, node_ids, edge_index, deg_emb, id_emb,
         W_self_0, W_neigh_0, b_0,
         W_self_1, W_neigh_1, b_1,
         W_self_2, W_neigh_2, b_2):
    i32 = jnp.int32
    degree = degree.astype(i32)
    node_ids = node_ids.astype(i32)
    src = edge_index[0].astype(i32)
    dst = edge_index[1].astype(i32)

    deg_idx = jnp.concatenate(
        [degree, jnp.zeros((_NP2 - _N,), i32)]).reshape(_NW * _GSTEPS, _CW)
    id_idx = jnp.concatenate(
        [node_ids, jnp.zeros((_NP2 - _N,), i32)]).reshape(_NW * _GSTEPS, _CW)
    src_p = jnp.concatenate(
        [src, jnp.zeros((_EPAD - _E,), i32)]).reshape(-1, _CW)
    dst_p = jnp.concatenate(
        [dst, jnp.full((_EPAD - _E,), _N, i32)]).reshape(-1, _CW)
    zeros_c = jnp.zeros((_CW, _CW), jnp.float32)
    ones_c = jnp.ones((_CW, _CW), jnp.float32)

    df, if_ = _get_emb_kernel()(deg_idx, id_idx, deg_emb, id_emb)

    p_df, p_if, p_deg = _make_agg(2, True)(df, if_, src_p, dst_p,
                                           zeros_c, ones_c)

    wsd = W_self_0[:, :_CW].T
    wsi = W_self_0[:, _CW:].T
    wnd = W_neigh_0[:, :_CW].T
    wni = W_neigh_0[:, _CW:].T
    h1c = _tc_layer0(df, if_, p_df, p_if, p_deg,
                     wsd, wsi, wnd, wni, b_0.reshape(1, _HID))

    p1c = _make_agg(4, False)(*h1c, src_p, dst_p, zeros_c, ones_c)

    ws1c = tuple(W_self_1[:, c * _CW:(c + 1) * _CW].T for c in range(4))
    wn1c = tuple(W_neigh_1[:, c * _CW:(c + 1) * _CW].T for c in range(4))
    hs, hn0, hn1 = _tc_layer12(h1c, p1c, p_deg, ws1c, wn1c,
                               b_1.reshape(1, _HID),
                               W_self_2.T, W_neigh_2.T,
                               b_2.reshape(1, _OUT))

    q0, q1 = _make_agg(2, False)(hn0, hn1, src_p, dst_p, zeros_c, ones_c)

    out = _tc_final(hs, q0, q1, p_deg)
    return out[:_N]


def kernel(degree, node_ids, edge_index, deg_emb, id_emb,
           W_self_0, W_neigh_0, b_0,
           W_self_1, W_neigh_1, b_1,
           W_self_2, W_neigh_2, b_2):
    return _run(degree, node_ids, edge_index, deg_emb, id_emb,
                W_self_0, W_neigh_0, b_0,
                W_self_1, W_neigh_1, b_1,
                W_self_2, W_neigh_2, b_2)
